# baseline (speedup 1.0000x reference)
import jax
import jax.numpy as jnp
from jax.experimental import pallas as pl
from jax.experimental.pallas import tpu as pltpu

_B, _C, _K, _P, _H, _W = 16, 64, 1024, 1024, 32, 32


def _vq_body(x, ea, esqh):
    e = ea[:, :_C]
    s = jax.lax.dot_general(
        e, x, (((1,), (0,)), ((), ())),
        preferred_element_type=jnp.float32) - esqh
    smax = jnp.max(s, axis=0, keepdims=True)
    oh = (s == smax).astype(jnp.float32)
    out = jax.lax.dot_general(
        ea, oh, (((0,), (0,)), ((), ())),
        preferred_element_type=jnp.float32)
    r = 1.0 / out[_C + 2:_C + 3]
    zq = out[:_C] * r
    ind_f = (out[_C:_C + 1] * 32.0 + out[_C + 1:_C + 2]) * r
    ind = (ind_f + 0.5).astype(jnp.int32)
    d = jnp.sum(x * x) - 2.0 * jnp.sum(smax)
    return zq, ind, d


def _vq_kernel(x_ref, ea_ref, zq_ref, ind_ref, dsum_ref):
    ea = ea_ref[...]
    e = ea[:, :_C]
    esqh = 0.5 * jnp.sum(e * e, axis=1, keepdims=True)
    acc = jnp.float32(0.0)
    for b in range(_B):
        x = x_ref[b].reshape(_C, _P)
        zq, ind, d = _vq_body(x, ea, esqh)
        zq_ref[b] = zq.reshape(_C, _H, _W)
        ind_ref[b] = ind.reshape(_H, _W)
        acc = acc + d
    dsum_ref[...] = (acc * (12.5 / (_B * _C * _P))).reshape(1, 1)


def kernel(z_e, embed_weight):
    B, C, H, W = z_e.shape
    K = embed_weight.shape[0]
    iota = jnp.arange(K, dtype=jnp.float32)[:, None]
    ea = jnp.concatenate(
        [embed_weight,
         jnp.floor(iota / 32.0),
         jnp.mod(iota, 32.0),
         jnp.ones((K, 1), jnp.float32)],
        axis=1)
    z_q_out, ind, diff11 = pl.pallas_call(
        _vq_kernel,
        in_specs=[
            pl.BlockSpec(memory_space=pltpu.VMEM),
            pl.BlockSpec(memory_space=pltpu.VMEM),
        ],
        out_specs=[
            pl.BlockSpec(memory_space=pltpu.VMEM),
            pl.BlockSpec(memory_space=pltpu.VMEM),
            pl.BlockSpec(memory_space=pltpu.VMEM),
        ],
        out_shape=[
            jax.ShapeDtypeStruct((B, C, H, W), jnp.float32),
            jax.ShapeDtypeStruct((B, H, W), jnp.int32),
            jax.ShapeDtypeStruct((1, 1), jnp.float32),
        ],
    )(z_e, ea)
    diff = diff11[0, 0]
    return (z_q_out, diff, ind)


# speedup vs baseline: 1.6642x; 1.6642x over previous
import jax
import jax.numpy as jnp
from jax.experimental import pallas as pl
from jax.experimental.pallas import tpu as pltpu

_B, _C, _K, _P = 16, 64, 1024, 1024


def _vq_body(x, ea, esqh):
    e = ea[:, :_C]
    s = jax.lax.dot_general(
        e, x, (((1,), (0,)), ((), ())),
        preferred_element_type=jnp.float32) - esqh
    smax = jnp.max(s, axis=0, keepdims=True)
    oh = (s == smax).astype(jnp.float32)
    out = jax.lax.dot_general(
        ea, oh, (((0,), (0,)), ((), ())),
        preferred_element_type=jnp.float32)
    r = 1.0 / out[_C + 2:_C + 3]
    zq = out[:_C] * r
    ind_f = (out[_C:_C + 1] * 32.0 + out[_C + 1:_C + 2]) * r
    ind = (ind_f + 0.5).astype(jnp.int32)
    d = jnp.sum(x * x) - 2.0 * jnp.sum(smax)
    return zq, ind, d


def _vq_kernel(x_ref, ea_ref, zq_ref, ind_ref, dsum_ref):
    ea = ea_ref[...]
    e = ea[:, :_C]
    esqh = 0.5 * jnp.sum(e * e, axis=1, keepdims=True)
    acc = jnp.float32(0.0)
    for b in range(_B):
        zq, ind, d = _vq_body(x_ref[b], ea, esqh)
        zq_ref[b] = zq
        ind_ref[b] = ind
        acc = acc + d
    dsum_ref[...] = (acc * (12.5 / (_B * _C * _P))).reshape(1, 1)


def kernel(z_e, embed_weight):
    B, C, H, W = z_e.shape
    K = embed_weight.shape[0]
    P = H * W
    x = z_e.reshape(B, C, P)
    iota = jnp.arange(K, dtype=jnp.float32)[:, None]
    ea = jnp.concatenate(
        [embed_weight,
         jnp.floor(iota / 32.0),
         jnp.mod(iota, 32.0),
         jnp.ones((K, 1), jnp.float32)],
        axis=1)
    zq, ind3, diff11 = pl.pallas_call(
        _vq_kernel,
        in_specs=[
            pl.BlockSpec(memory_space=pltpu.VMEM),
            pl.BlockSpec(memory_space=pltpu.VMEM),
        ],
        out_specs=[
            pl.BlockSpec(memory_space=pltpu.VMEM),
            pl.BlockSpec(memory_space=pltpu.VMEM),
            pl.BlockSpec(memory_space=pltpu.VMEM),
        ],
        out_shape=[
            jax.ShapeDtypeStruct((B, C, P), jnp.float32),
            jax.ShapeDtypeStruct((B, 1, P), jnp.int32),
            jax.ShapeDtypeStruct((1, 1), jnp.float32),
        ],
    )(x, ea)
    z_q_out = zq.reshape(B, C, H, W)
    ind = ind3.reshape(B, H, W)
    diff = diff11[0, 0]
    return (z_q_out, diff, ind)
